# baseline (speedup 1.0000x reference)
import functools

import jax
import jax.numpy as jnp
from jax import lax
from jax.experimental import pallas as pl
from jax.experimental.pallas import tpu as pltpu
from jax.experimental.pallas import tpu_sc as plsc

S, C, NE, DFF = 2048, 768, 8, 3072
BLK = 128
NBLK_MAX = S // BLK + NE
P = NBLK_MAX * BLK
DBLK = 1536
ND = DFF // DBLK
NC, NS = 2, 16
NW = NC * NS
BPW = P // NW


def _router_body(x_ref, wg_ref, inv_ref, w_ref, nblk_ref, boff_ref, aux_ref):
    xv = x_ref[...]
    logits = jnp.dot(xv, wg_ref[...], preferred_element_type=jnp.float32)
    m = jnp.max(logits, axis=-1, keepdims=True)
    p = jnp.exp(logits - m)
    probs = p / jnp.sum(p, axis=-1, keepdims=True)
    w = jnp.max(probs, axis=-1, keepdims=True)
    idx = jnp.argmax(probs, axis=-1).astype(jnp.int32).reshape(S, 1)
    w_ref[...] = jnp.broadcast_to(w, (S, 128))
    oh = (idx == lax.broadcasted_iota(jnp.int32, (1, NE), 1)
          ).astype(jnp.float32)
    G, GS = 16, 128
    lt_incl = (lax.broadcasted_iota(jnp.int32, (GS, GS), 0)
               >= lax.broadcasted_iota(jnp.int32, (GS, GS), 1)
               ).astype(jnp.float32)
    pieces, tots = [], []
    for g in range(G):
        blk = lax.slice(oh, (g * GS, 0), ((g + 1) * GS, NE))
        pg = jnp.dot(lt_incl, blk, preferred_element_type=jnp.float32)
        pieces.append(pg)
        tots.append(lax.slice(pg, (GS - 1, 0), (GS, NE)))
    totals = jnp.concatenate(tots, axis=0)
    lt_excl = (lax.broadcasted_iota(jnp.int32, (G, G), 0)
               > lax.broadcasted_iota(jnp.int32, (G, G), 1)
               ).astype(jnp.float32)
    cumg = jnp.dot(lt_excl, totals, preferred_element_type=jnp.float32)
    pref = jnp.concatenate(
        [pieces[g] + lax.slice(cumg, (g, 0), (g + 1, NE)) for g in range(G)],
        axis=0)
    cnt = lax.slice(pref, (S - 1, 0), (S, NE))
    nblk = (cnt.astype(jnp.int32) + (BLK - 1)) // BLK
    ut_excl8 = (lax.broadcasted_iota(jnp.int32, (NE, NE), 0)
                < lax.broadcasted_iota(jnp.int32, (NE, NE), 1)
                ).astype(jnp.float32)
    boff = jnp.dot(nblk.astype(jnp.float32), ut_excl8,
                   preferred_element_type=jnp.float32)
    rank = jnp.sum(pref * oh, axis=1, keepdims=True) - 1.0
    boff_tok = jnp.sum(oh * boff, axis=1, keepdims=True)
    inv_ref[...] = (boff_tok * BLK + rank).astype(jnp.int32)
    nblk_ref[...] = nblk
    boff_ref[...] = boff.astype(jnp.int32)
    density = jnp.mean(oh, axis=0, keepdims=True)
    proxy = jnp.mean(probs, axis=0, keepdims=True)
    aux_ref[...] = (jnp.sum(density * proxy) * NE).reshape(1, 1)


def _router(x_flat, Wg):
    return pl.pallas_call(
        _router_body,
        out_shape=(
            jax.ShapeDtypeStruct((S, 1), jnp.int32),
            jax.ShapeDtypeStruct((S, 128), jnp.float32),
            jax.ShapeDtypeStruct((1, NE), jnp.int32),
            jax.ShapeDtypeStruct((1, NE), jnp.int32),
            jax.ShapeDtypeStruct((1, 1), jnp.float32),
        ),
    )(x_flat, Wg)


def _sc_dispatch(x_flat, w_rep, inv_ids):
    mesh = plsc.VectorSubcoreMesh(core_axis_name="c", subcore_axis_name="s")
    tpw = S // NW

    @functools.partial(
        pl.kernel,
        out_type=(
            jax.ShapeDtypeStruct((P, C), jnp.float32),
            jax.ShapeDtypeStruct((P, 128), jnp.float32),
        ),
        mesh=mesh,
        scratch_types=[
            pltpu.VMEM((tpw,), jnp.int32),
            pltpu.VMEM((tpw, C), jnp.float32),
            pltpu.VMEM((tpw, 128), jnp.float32),
            pltpu.SemaphoreType.DMA,
            pltpu.SemaphoreType.DMA,
        ],
    )
    def k(x_hbm, wrep_hbm, inv_hbm, xs_out, ws_out, idx_v, rows_v, wrow_v,
          sem, sem2):
        wid = lax.axis_index("s") * NC + lax.axis_index("c")
        base = wid * tpw
        pltpu.sync_copy(inv_hbm.at[pl.ds(base, tpw)], idx_v)
        pltpu.sync_copy(x_hbm.at[pl.ds(base, tpw)], rows_v)
        pltpu.sync_copy(wrep_hbm.at[pl.ds(base, tpw)], wrow_v)
        cp1 = pltpu.async_copy(rows_v, xs_out.at[idx_v], sem)
        cp2 = pltpu.async_copy(wrow_v, ws_out.at[idx_v], sem2)
        cp1.wait()
        cp2.wait()

    return k(x_flat, w_rep, inv_ids)


def _gemm_body(nblk_ref, boff_ref, xs_ref, ws_ref, w1_ref, w3_ref, w2_ref,
               out_ref):
    e = pl.program_id(0)
    d = pl.program_id(1)
    base_blk = boff_ref[e]
    n = nblk_ref[e]
    W1c = w1_ref[0]
    W3c = w3_ref[0]
    W2c = w2_ref[0]

    def step(i, carry):
        r0 = pl.multiple_of((base_blk + i) * BLK, BLK)
        xb = xs_ref[pl.ds(r0, BLK), :]
        wb = ws_ref[pl.ds(r0, BLK), :]
        g = jnp.dot(xb, W1c, preferred_element_type=jnp.float32,
                    precision=lax.Precision.DEFAULT)
        u = jnp.dot(xb, W3c, preferred_element_type=jnp.float32,
                    precision=lax.Precision.DEFAULT)
        h = (g * jax.nn.sigmoid(g)) * u
        part = jnp.dot(h, W2c, preferred_element_type=jnp.float32,
                       precision=lax.Precision.DEFAULT) * wb

        @pl.when(d == 0)
        def _():
            out_ref[pl.ds(r0, BLK), :] = part

        @pl.when(d != 0)
        def _():
            out_ref[pl.ds(r0, BLK), :] += part

        return carry

    lax.fori_loop(0, n, step, 0)


def _grouped_gemm(nblk, blkoff, xs, ws_col, W1, W3, W2):
    grid_spec = pltpu.PrefetchScalarGridSpec(
        num_scalar_prefetch=2,
        grid=(NE, ND),
        in_specs=[
            pl.BlockSpec((P, C), lambda e, d, nb, bo: (0, 0)),
            pl.BlockSpec((P, 1), lambda e, d, nb, bo: (0, 0)),
            pl.BlockSpec((1, C, DBLK), lambda e, d, nb, bo: (e, 0, d)),
            pl.BlockSpec((1, C, DBLK), lambda e, d, nb, bo: (e, 0, d)),
            pl.BlockSpec((1, DBLK, C), lambda e, d, nb, bo: (e, d, 0)),
        ],
        out_specs=pl.BlockSpec((P, C), lambda e, d, nb, bo: (0, 0)),
    )
    return pl.pallas_call(
        _gemm_body,
        grid_spec=grid_spec,
        out_shape=jax.ShapeDtypeStruct((P, C), jnp.float32),
        compiler_params=pltpu.CompilerParams(
            dimension_semantics=("arbitrary", "arbitrary"),
            vmem_limit_bytes=63 * 1024 * 1024),
    )(nblk, blkoff, xs, ws_col, W1, W3, W2)


def _sc_scatter(ys, scatter_ids):
    mesh = plsc.VectorSubcoreMesh(core_axis_name="c", subcore_axis_name="s")

    @functools.partial(
        pl.kernel,
        out_type=jax.ShapeDtypeStruct((S + P, C), jnp.float32),
        mesh=mesh,
        scratch_types=[
            pltpu.VMEM((BPW,), jnp.int32),
            pltpu.VMEM((BPW, C), jnp.float32),
            pltpu.SemaphoreType.DMA,
        ],
    )
    def k(ys_hbm, sidx_hbm, out_hbm, idx_v, rows_v, sem):
        wid = lax.axis_index("s") * NC + lax.axis_index("c")
        base = wid * BPW
        pltpu.sync_copy(sidx_hbm.at[pl.ds(base, BPW)], idx_v)
        pltpu.sync_copy(ys_hbm.at[pl.ds(base, BPW)], rows_v)
        pltpu.async_copy(rows_v, out_hbm.at[idx_v], sem).wait()

    return k(ys, scatter_ids)


def kernel(x, Wg, W1, W3, W2):
    b, t, c = x.shape
    x_flat = x.reshape(S, C)
    inv2, w2, nblk2, boff2, aux2 = _router(x_flat, Wg)
    inv_ids = inv2[:, 0]
    scatter_ids = (S + jnp.arange(P, dtype=jnp.int32)).at[inv_ids].set(
        jnp.arange(S, dtype=jnp.int32))
    xs, ws_rep = _sc_dispatch(x_flat, w2, inv_ids)
    ys = _grouped_gemm(nblk2[0], boff2[0], xs, ws_rep[:, :1], W1, W3, W2)
    y_big = _sc_scatter(ys, scatter_ids)
    y = y_big[:S].reshape(b, t, c)
    return (y, aux2[0, 0])


# speedup vs baseline: 1.0054x; 1.0054x over previous
import functools

import jax
import jax.numpy as jnp
from jax import lax
from jax.experimental import pallas as pl
from jax.experimental.pallas import tpu as pltpu
from jax.experimental.pallas import tpu_sc as plsc

S, C, NE, DFF = 2048, 768, 8, 3072
BLK = 128
NBLK_MAX = S // BLK + NE
P = NBLK_MAX * BLK
DBLK = 1536
ND = DFF // DBLK
NC, NS = 2, 16
NW = NC * NS
BPW = P // NW


def _router_body(x_ref, wg_ref, inv_ref, w_ref, nblk_ref, boff_ref, aux_ref):
    xv = x_ref[...]
    logits = jnp.dot(xv, wg_ref[...], preferred_element_type=jnp.float32)
    m = jnp.max(logits, axis=-1, keepdims=True)
    p = jnp.exp(logits - m)
    probs = p / jnp.sum(p, axis=-1, keepdims=True)
    w = jnp.max(probs, axis=-1, keepdims=True)
    idx = jnp.argmax(probs, axis=-1).astype(jnp.int32).reshape(S, 1)
    w_ref[...] = jnp.broadcast_to(w, (S, 128))
    oh = (idx == lax.broadcasted_iota(jnp.int32, (1, NE), 1)
          ).astype(jnp.float32)
    G, GS = 16, 128
    lt_incl = (lax.broadcasted_iota(jnp.int32, (GS, GS), 0)
               >= lax.broadcasted_iota(jnp.int32, (GS, GS), 1)
               ).astype(jnp.float32)
    pieces, tots = [], []
    for g in range(G):
        blk = lax.slice(oh, (g * GS, 0), ((g + 1) * GS, NE))
        pg = jnp.dot(lt_incl, blk, preferred_element_type=jnp.float32)
        pieces.append(pg)
        tots.append(lax.slice(pg, (GS - 1, 0), (GS, NE)))
    totals = jnp.concatenate(tots, axis=0)
    lt_excl = (lax.broadcasted_iota(jnp.int32, (G, G), 0)
               > lax.broadcasted_iota(jnp.int32, (G, G), 1)
               ).astype(jnp.float32)
    cumg = jnp.dot(lt_excl, totals, preferred_element_type=jnp.float32)
    pref = jnp.concatenate(
        [pieces[g] + lax.slice(cumg, (g, 0), (g + 1, NE)) for g in range(G)],
        axis=0)
    cnt = lax.slice(pref, (S - 1, 0), (S, NE))
    nblk = (cnt.astype(jnp.int32) + (BLK - 1)) // BLK
    ut_excl8 = (lax.broadcasted_iota(jnp.int32, (NE, NE), 0)
                < lax.broadcasted_iota(jnp.int32, (NE, NE), 1)
                ).astype(jnp.float32)
    boff = jnp.dot(nblk.astype(jnp.float32), ut_excl8,
                   preferred_element_type=jnp.float32)
    rank = jnp.sum(pref * oh, axis=1, keepdims=True) - 1.0
    boff_tok = jnp.sum(oh * boff, axis=1, keepdims=True)
    inv_ref[...] = (boff_tok * BLK + rank).astype(jnp.int32)
    nblk_ref[...] = nblk
    boff_ref[...] = boff.astype(jnp.int32)
    density = jnp.mean(oh, axis=0, keepdims=True)
    proxy = jnp.mean(probs, axis=0, keepdims=True)
    aux_ref[...] = (jnp.sum(density * proxy) * NE).reshape(1, 1)


def _router(x_flat, Wg):
    return pl.pallas_call(
        _router_body,
        out_shape=(
            jax.ShapeDtypeStruct((S, 1), jnp.int32),
            jax.ShapeDtypeStruct((S, 128), jnp.float32),
            jax.ShapeDtypeStruct((1, NE), jnp.int32),
            jax.ShapeDtypeStruct((1, NE), jnp.int32),
            jax.ShapeDtypeStruct((1, 1), jnp.float32),
        ),
    )(x_flat, Wg)


def _sc_dispatch(x_flat, w_rep, inv_ids):
    mesh = plsc.VectorSubcoreMesh(core_axis_name="c", subcore_axis_name="s")
    tpw = S // NW

    @functools.partial(
        pl.kernel,
        out_type=(
            jax.ShapeDtypeStruct((P, C), jnp.float32),
            jax.ShapeDtypeStruct((P, 128), jnp.float32),
        ),
        mesh=mesh,
        scratch_types=[
            pltpu.VMEM((tpw,), jnp.int32),
            pltpu.VMEM((tpw, C), jnp.float32),
            pltpu.VMEM((tpw, 128), jnp.float32),
            pltpu.SemaphoreType.DMA,
            pltpu.SemaphoreType.DMA,
        ],
    )
    def k(x_hbm, wrep_hbm, inv_hbm, xs_out, ws_out, idx_v, rows_v, wrow_v,
          sem, sem2):
        wid = lax.axis_index("s") * NC + lax.axis_index("c")
        base = wid * tpw
        pltpu.sync_copy(inv_hbm.at[pl.ds(base, tpw)], idx_v)
        pltpu.sync_copy(x_hbm.at[pl.ds(base, tpw)], rows_v)
        pltpu.sync_copy(wrep_hbm.at[pl.ds(base, tpw)], wrow_v)
        cp1 = pltpu.async_copy(rows_v, xs_out.at[idx_v], sem)
        cp2 = pltpu.async_copy(wrow_v, ws_out.at[idx_v], sem2)
        cp1.wait()
        cp2.wait()

    return k(x_flat, w_rep, inv_ids)


def _gemm_body(nblk_ref, boff_ref, xs_ref, ws_ref, w1_ref, w3_ref, w2_ref,
               out_ref):
    e = pl.program_id(0)
    d = pl.program_id(1)
    base_blk = boff_ref[e]
    n = nblk_ref[e]
    W1c = w1_ref[0]
    W3c = w3_ref[0]
    W2c = w2_ref[0]

    def step(i, carry):
        r0 = pl.multiple_of((base_blk + i) * BLK, BLK)
        xb = xs_ref[pl.ds(r0, BLK), :]
        wb = ws_ref[pl.ds(r0, BLK), :]
        g = jnp.dot(xb, W1c, preferred_element_type=jnp.float32)
        u = jnp.dot(xb, W3c, preferred_element_type=jnp.float32)
        h = (g * jax.nn.sigmoid(g)) * u
        part = jnp.dot(h, W2c, preferred_element_type=jnp.float32) * wb

        @pl.when(d == 0)
        def _():
            out_ref[pl.ds(r0, BLK), :] = part

        @pl.when(d != 0)
        def _():
            out_ref[pl.ds(r0, BLK), :] += part

        return carry

    lax.fori_loop(0, n, step, 0)


def _grouped_gemm(nblk, blkoff, xs, ws_col, W1, W3, W2):
    grid_spec = pltpu.PrefetchScalarGridSpec(
        num_scalar_prefetch=2,
        grid=(NE, ND),
        in_specs=[
            pl.BlockSpec((P, C), lambda e, d, nb, bo: (0, 0)),
            pl.BlockSpec((P, 1), lambda e, d, nb, bo: (0, 0)),
            pl.BlockSpec((1, C, DBLK), lambda e, d, nb, bo: (e, 0, d)),
            pl.BlockSpec((1, C, DBLK), lambda e, d, nb, bo: (e, 0, d)),
            pl.BlockSpec((1, DBLK, C), lambda e, d, nb, bo: (e, d, 0)),
        ],
        out_specs=pl.BlockSpec((P, C), lambda e, d, nb, bo: (0, 0)),
    )
    return pl.pallas_call(
        _gemm_body,
        grid_spec=grid_spec,
        out_shape=jax.ShapeDtypeStruct((P, C), jnp.float32),
        compiler_params=pltpu.CompilerParams(
            dimension_semantics=("arbitrary", "arbitrary"),
            vmem_limit_bytes=63 * 1024 * 1024),
    )(nblk, blkoff, xs, ws_col, W1, W3, W2)


def _sc_scatter(ys, scatter_ids):
    mesh = plsc.VectorSubcoreMesh(core_axis_name="c", subcore_axis_name="s")

    @functools.partial(
        pl.kernel,
        out_type=jax.ShapeDtypeStruct((S + P, C), jnp.float32),
        mesh=mesh,
        scratch_types=[
            pltpu.VMEM((BPW,), jnp.int32),
            pltpu.VMEM((BPW, C), jnp.float32),
            pltpu.SemaphoreType.DMA,
        ],
    )
    def k(ys_hbm, sidx_hbm, out_hbm, idx_v, rows_v, sem):
        wid = lax.axis_index("s") * NC + lax.axis_index("c")
        base = wid * BPW
        pltpu.sync_copy(sidx_hbm.at[pl.ds(base, BPW)], idx_v)
        pltpu.sync_copy(ys_hbm.at[pl.ds(base, BPW)], rows_v)
        pltpu.async_copy(rows_v, out_hbm.at[idx_v], sem).wait()

    return k(ys, scatter_ids)


def kernel(x, Wg, W1, W3, W2):
    b, t, c = x.shape
    x_flat = x.reshape(S, C)
    inv2, w2, nblk2, boff2, aux2 = _router(x_flat, Wg)
    inv_ids = inv2[:, 0]
    scatter_ids = (S + jnp.arange(P, dtype=jnp.int32)).at[inv_ids].set(
        jnp.arange(S, dtype=jnp.int32))
    xs, ws_rep = _sc_dispatch(x_flat, w2, inv_ids)
    ys = _grouped_gemm(nblk2[0], boff2[0], xs, ws_rep[:, :1], W1, W3, W2)
    y_big = _sc_scatter(ys, scatter_ids)
    y = y_big[:S].reshape(b, t, c)
    return (y, aux2[0, 0])
